# baseline (speedup 1.0000x reference)
import functools

import jax
import jax.numpy as jnp
from jax import lax
from jax.experimental import pallas as pl
from jax.experimental.pallas import tpu as pltpu
from jax.experimental.pallas import tpu_sc as plsc

N = 10000
E = 320000
D = 128
H = D // 2
SLOPE = 0.2

NC = 2
NS = 16
C = 256
G = 80
E_PAD = NS * G * C
RPT = 632
N_PAD = NS * RPT


def _spmm_body(cols_hbm, rows_hbm, vals_hbm, embeds_hbm, out_hbm,
               c0, c1, c2, c3, r0, r1, r2, r3, v0, v1, v2, v3,
               g0, g1, ibuf, emb_sh, acc,
               sm0, sm1, sm2, sm3, sge, sg0, sg1, ss0, ss1):
    cbufs = (c0, c1, c2, c3)
    rbufs = (r0, r1, r2, r3)
    vbufs = (v0, v1, v2, v3)
    gbs = (g0, g1)
    sms = (sm0, sm1, sm2, sm3)
    sgs = (sg0, sg1)
    sss = (ss0, ss1)

    cid = lax.axis_index("c")
    sid = lax.axis_index("s")
    ebase = sid * G * C

    def load_meta(m4, c):
        sl = pl.ds(ebase + c * C, C)
        pltpu.async_copy(cols_hbm.at[sl], cbufs[m4], sms[m4])
        pltpu.async_copy(rows_hbm.at[sl], rbufs[m4], sms[m4])
        pltpu.async_copy(vals_hbm.at[sl], vbufs[m4], sms[m4])

    def wait_meta(m4):
        sl = pl.ds(0, C)
        pltpu.make_async_copy(cols_hbm.at[sl], cbufs[m4], sms[m4]).wait()
        pltpu.make_async_copy(rows_hbm.at[sl], rbufs[m4], sms[m4]).wait()
        pltpu.make_async_copy(vals_hbm.at[sl], vbufs[m4], sms[m4]).wait()

    def gather(m4, b2):
        pltpu.async_copy(emb_sh.at[cbufs[m4]], gbs[b2], sgs[b2])

    def wait_gather(m4, b2):
        pltpu.make_async_copy(emb_sh.at[cbufs[m4]], gbs[b2],
                              sgs[b2]).wait()

    def scale(m4, b2):
        @plsc.parallel_loop(0, C // 16, unroll=2)
        def _grp(q):
            vvec = vbufs[m4][pl.ds(q * 16, 16)]
            for j in range(16):
                s = vvec.at[jnp.full((16,), j, jnp.int32)].get(
                    mode="promise_in_bounds")
                e = q * 16 + j
                for f in range(H // 16):
                    w = pl.ds(f * 16, 16)
                    gbs[b2][e, w] = gbs[b2][e, w] * s

    def scatter(m4, b2):
        pltpu.async_copy(gbs[b2], acc.at[rbufs[m4]], sss[b2], add=True)

    def wait_scatter(m4, b2):
        pltpu.make_async_copy(gbs[b2], acc.at[rbufs[m4]], sss[b2]).wait()

    def chunk(m4, b2, c, do_ws, do_load, do_next):
        if do_ws:
            wait_scatter((m4 + 3) % 4, (b2 + 1) % 2)
        if do_load:
            load_meta((m4 + 3) % 4, c + 3)
        if do_next:
            wait_meta((m4 + 1) % 4)
            gather((m4 + 1) % 4, (b2 + 1) % 2)
        wait_gather(m4, b2)
        scale(m4, b2)
        scatter(m4, b2)

    load_meta(0, 0)
    load_meta(1, 1)
    load_meta(2, 2)

    def _zrow(r, carry):
        for f in range(H // 16):
            g0[r, pl.ds(f * 16, 16)] = jnp.zeros((16,), jnp.float32)
        return carry
    lax.fori_loop(0, 128, _zrow, 0)
    zb = g0.at[pl.ds(0, 128)]
    zbase = sid * RPT
    for k in range(RPT // 128):
        pltpu.sync_copy(zb, acc.at[pl.ds(zbase + k * 128, 128)])
    pltpu.sync_copy(g0.at[pl.ds(0, RPT % 128)],
                    acc.at[pl.ds(zbase + (RPT // 128) * 128, RPT % 128)])

    nbase = sid * (N // NS)

    def _ib(q, carry):
        w = pl.ds(q * 16, 16)
        ibuf[w] = (lax.iota(jnp.int32, 16)
                   + jnp.full((16,), nbase + q * 16, jnp.int32)) * 2 + cid
        return carry
    lax.fori_loop(0, 40, _ib, 0)
    for off, sz in ((0, 256), (256, 256), (512, 113)):
        pltpu.async_copy(embeds_hbm.at[ibuf.at[pl.ds(off, sz)]],
                         g0.at[pl.ds(0, sz)], sge).wait()
        pltpu.sync_copy(g0.at[pl.ds(0, sz)],
                        emb_sh.at[pl.ds(nbase + off, sz)])
    plsc.subcore_barrier()

    wait_meta(0)
    gather(0, 0)

    chunk(0, 0, 0, do_ws=False, do_load=True, do_next=True)

    def body4(i, carry):
        for m in range(4):
            c = 1 + i * 4 + m
            chunk((1 + m) % 4, (1 + m) % 2, c,
                  do_ws=True, do_load=True, do_next=True)
        return carry
    lax.fori_loop(0, (G - 4) // 4, body4, 0)

    chunk((G - 3) % 4, (G - 3) % 2, G - 3, do_ws=True, do_load=False,
          do_next=True)
    chunk((G - 2) % 4, (G - 2) % 2, G - 2, do_ws=True, do_load=False,
          do_next=True)
    chunk((G - 1) % 4, (G - 1) % 2, G - 1, do_ws=True, do_load=False,
          do_next=False)
    wait_scatter((G - 1) % 4, (G - 1) % 2)

    plsc.subcore_barrier()

    obase = sid * RPT

    def flush(sizes):
        off = 0
        for sz in sizes:
            pltpu.sync_copy(acc.at[pl.ds(obase + off, sz)],
                            g0.at[pl.ds(0, sz)])

            @plsc.parallel_loop(0, sz)
            def _lk(r):
                for f in range(H // 16):
                    w = pl.ds(f * 16, 16)
                    x = g0[r, w]
                    g0[r, w] = jnp.where(x > 0, x, SLOPE * x)
            pltpu.sync_copy(g0.at[pl.ds(0, sz)],
                            out_hbm.at[pl.ds(obase + off, sz), cid])
            off += sz

    @pl.when(sid < NS - 1)
    def _full():
        flush((128, 128, 128, 128, 120))

    @pl.when(sid == NS - 1)
    def _last():
        flush((128, 128, 128, 128, 8))


_spmm_sc = functools.partial(
    pl.kernel,
    out_type=jax.ShapeDtypeStruct((N, NC, H), jnp.float32),
    mesh=plsc.VectorSubcoreMesh(core_axis_name="c", subcore_axis_name="s"),
    compiler_params=pltpu.CompilerParams(use_tc_tiling_on_sc=False),
    scratch_types=(
        [pltpu.VMEM((C,), jnp.int32) for _ in range(4)]
        + [pltpu.VMEM((C,), jnp.int32) for _ in range(4)]
        + [pltpu.VMEM((C,), jnp.float32) for _ in range(4)]
        + [pltpu.VMEM((C, H), jnp.float32) for _ in range(2)]
        + [pltpu.VMEM((640,), jnp.int32)]
        + [pltpu.VMEM_SHARED((N, H), jnp.float32)]
        + [pltpu.VMEM_SHARED((N_PAD, H), jnp.float32)]
        + [pltpu.SemaphoreType.DMA for _ in range(9)]
    ),
)(_spmm_body)


def kernel(adj_indices, adj_values, embeds):
    rows = adj_indices[0].astype(jnp.int32)
    cols = adj_indices[1].astype(jnp.int32)
    vals = adj_values.astype(jnp.float32)
    pad = E_PAD - E
    rows = jnp.pad(rows, (0, pad))
    cols = jnp.pad(cols, (0, pad))
    vals = jnp.pad(vals, (0, pad))
    emb2 = embeds.reshape(2 * N, H)
    partials = _spmm_sc(cols, rows, vals, emb2)
    return partials.reshape(N, D)


# speedup vs baseline: 1.0956x; 1.0956x over previous
import functools

import jax
import jax.numpy as jnp
from jax import lax
from jax.experimental import pallas as pl
from jax.experimental.pallas import tpu as pltpu
from jax.experimental.pallas import tpu_sc as plsc

N = 10000
E = 320000
D = 128
H = D // 2
SLOPE = 0.2

NC = 2
NS = 16
C = 256
G = 80
E_PAD = NS * G * C
RPT = 632
N_PAD = NS * RPT


def _spmm_body(cols_hbm, rows_hbm, vals_hbm, embeds_hbm, out_hbm,
               c0, c1, c2, c3, r0, r1, r2, r3, v0, v1, v2, v3,
               g0, g1, ibuf, emb_sh, acc,
               sm0, sm1, sm2, sm3, sge, sg0, sg1, ss0, ss1):
    cbufs = (c0, c1, c2, c3)
    rbufs = (r0, r1, r2, r3)
    vbufs = (v0, v1, v2, v3)
    gbs = (g0, g1)
    sms = (sm0, sm1, sm2, sm3)
    sgs = (sg0, sg1)
    sss = (ss0, ss1)

    cid = lax.axis_index("c")
    sid = lax.axis_index("s")
    ebase = sid * G * C

    def load_meta(m4, c):
        sl = pl.ds(ebase + c * C, C)
        pltpu.async_copy(cols_hbm.at[sl], cbufs[m4], sms[m4])
        pltpu.async_copy(rows_hbm.at[sl], rbufs[m4], sms[m4])
        pltpu.async_copy(vals_hbm.at[sl], vbufs[m4], sms[m4])

    def wait_meta(m4):
        sl = pl.ds(0, C)
        pltpu.make_async_copy(cols_hbm.at[sl], cbufs[m4], sms[m4]).wait()
        pltpu.make_async_copy(rows_hbm.at[sl], rbufs[m4], sms[m4]).wait()
        pltpu.make_async_copy(vals_hbm.at[sl], vbufs[m4], sms[m4]).wait()

    def gather(m4, b2):
        pltpu.async_copy(emb_sh.at[cbufs[m4]], gbs[b2], sgs[b2])

    def wait_gather(m4, b2):
        pltpu.make_async_copy(emb_sh.at[cbufs[m4]], gbs[b2],
                              sgs[b2]).wait()

    def scale(m4, b2):
        @plsc.parallel_loop(0, C // 16, unroll=2)
        def _grp(q):
            vvec = vbufs[m4][pl.ds(q * 16, 16)]
            for j in range(16):
                s = vvec.at[jnp.full((16,), j, jnp.int32)].get(
                    mode="promise_in_bounds")
                e = q * 16 + j
                for f in range(H // 16):
                    w = pl.ds(f * 16, 16)
                    gbs[b2][e, w] = gbs[b2][e, w] * s

    def scatter(m4, b2):
        pltpu.async_copy(gbs[b2], acc.at[rbufs[m4]], sss[b2], add=True)

    def wait_scatter(m4, b2):
        pltpu.make_async_copy(gbs[b2], acc.at[rbufs[m4]], sss[b2]).wait()

    def chunk(m4, b2, c, do_ws, do_load, do_next):
        if do_ws:
            wait_scatter((m4 + 3) % 4, (b2 + 1) % 2)
        if do_load:
            load_meta((m4 + 3) % 4, c + 3)
        if do_next:
            wait_meta((m4 + 1) % 4)
            gather((m4 + 1) % 4, (b2 + 1) % 2)
        wait_gather(m4, b2)
        scale(m4, b2)
        scatter(m4, b2)

    load_meta(0, 0)
    load_meta(1, 1)
    load_meta(2, 2)

    def _zrow(r, carry):
        for f in range(H // 16):
            g0[r, pl.ds(f * 16, 16)] = jnp.zeros((16,), jnp.float32)
        return carry
    lax.fori_loop(0, 128, _zrow, 0)
    zb = g0.at[pl.ds(0, 128)]
    zbase = sid * RPT
    for k in range(RPT // 128):
        pltpu.sync_copy(zb, acc.at[pl.ds(zbase + k * 128, 128)])
    pltpu.sync_copy(g0.at[pl.ds(0, RPT % 128)],
                    acc.at[pl.ds(zbase + (RPT // 128) * 128, RPT % 128)])

    nbase = sid * (N // NS)

    def _ib(q, carry):
        w = pl.ds(q * 16, 16)
        ibuf[w] = (lax.iota(jnp.int32, 16)
                   + jnp.full((16,), nbase + q * 16, jnp.int32)) * 2 + cid
        return carry
    lax.fori_loop(0, 40, _ib, 0)
    for off, sz in ((0, 256), (256, 256), (512, 113)):
        pltpu.async_copy(embeds_hbm.at[ibuf.at[pl.ds(off, sz)]],
                         g0.at[pl.ds(0, sz)], sge).wait()
        pltpu.sync_copy(g0.at[pl.ds(0, sz)],
                        emb_sh.at[pl.ds(nbase + off, sz)])
    plsc.subcore_barrier()

    wait_meta(0)
    gather(0, 0)

    chunk(0, 0, 0, do_ws=False, do_load=True, do_next=True)

    def body4(i, carry):
        for m in range(4):
            c = 1 + i * 4 + m
            chunk((1 + m) % 4, (1 + m) % 2, c,
                  do_ws=True, do_load=True, do_next=True)
        return carry
    lax.fori_loop(0, (G - 4) // 4, body4, 0)

    chunk((G - 3) % 4, (G - 3) % 2, G - 3, do_ws=True, do_load=False,
          do_next=True)
    chunk((G - 2) % 4, (G - 2) % 2, G - 2, do_ws=True, do_load=False,
          do_next=True)
    chunk((G - 1) % 4, (G - 1) % 2, G - 1, do_ws=True, do_load=False,
          do_next=False)
    wait_scatter((G - 1) % 4, (G - 1) % 2)

    plsc.subcore_barrier()

    obase = sid * RPT
    pltpu.sync_copy(acc.at[pl.ds(obase, RPT)],
                    out_hbm.at[cid, pl.ds(obase, RPT)])


_spmm_sc = functools.partial(
    pl.kernel,
    out_type=jax.ShapeDtypeStruct((NC, N_PAD, H), jnp.float32),
    mesh=plsc.VectorSubcoreMesh(core_axis_name="c", subcore_axis_name="s"),
    compiler_params=pltpu.CompilerParams(use_tc_tiling_on_sc=False),
    scratch_types=(
        [pltpu.VMEM((C,), jnp.int32) for _ in range(4)]
        + [pltpu.VMEM((C,), jnp.int32) for _ in range(4)]
        + [pltpu.VMEM((C,), jnp.float32) for _ in range(4)]
        + [pltpu.VMEM((C, H), jnp.float32) for _ in range(2)]
        + [pltpu.VMEM((640,), jnp.int32)]
        + [pltpu.VMEM_SHARED((N, H), jnp.float32)]
        + [pltpu.VMEM_SHARED((N_PAD, H), jnp.float32)]
        + [pltpu.SemaphoreType.DMA for _ in range(9)]
    ),
)(_spmm_body)


def _combine_body(p_ref, o_ref):
    o_ref[:, :H] = jnp.where(p_ref[0] > 0, p_ref[0], SLOPE * p_ref[0])
    o_ref[:, H:] = jnp.where(p_ref[1] > 0, p_ref[1], SLOPE * p_ref[1])


def _combine(partials):
    bn = 1000
    return pl.pallas_call(
        _combine_body,
        out_shape=jax.ShapeDtypeStruct((N, D), jnp.float32),
        grid=(N // bn,),
        in_specs=[pl.BlockSpec((NC, bn, H), lambda i: (0, i, 0))],
        out_specs=pl.BlockSpec((bn, D), lambda i: (i, 0)),
    )(partials)


def kernel(adj_indices, adj_values, embeds):
    rows = adj_indices[0].astype(jnp.int32)
    cols = adj_indices[1].astype(jnp.int32)
    vals = adj_values.astype(jnp.float32)
    pad = E_PAD - E
    rows = jnp.pad(rows, (0, pad))
    cols = jnp.pad(cols, (0, pad))
    vals = jnp.pad(vals, (0, pad))
    emb2 = embeds.reshape(2 * N, H)
    partials = _spmm_sc(cols, rows, vals, emb2)
    return _combine(partials)
